# baseline (speedup 1.0000x reference)
import functools

import jax
import jax.numpy as jnp
from jax import lax
from jax.experimental import pallas as pl
from jax.experimental.pallas import tpu as pltpu
from jax.experimental.pallas import tpu_sc as plsc

NC = 2
NS = 16
L = 16
K = 2048
NB = 8
NPIX = 512 * 512
TILES_PER_IMG = (NC * NS) // NB
WORDS_PER_IMG = NPIX // 2
WCHUNK = WORDS_PER_IMG // TILES_PER_IMG
WPIECE = 4096
NPIECE = WCHUNK // WPIECE
VEC_PER_PIECE = WPIECE // L
HB = 2 * K
UNROLL = 16

IMG_H = 512
IMG_W = 512


def _pack_body(p_ref, t_ref, o_ref):
    p = p_ref[...]
    t = t_ref[...]
    e = jnp.abs(t.astype(jnp.float32) - p)
    q = jnp.minimum((e * float(K)).astype(jnp.int32), K - 1)
    a = t * K + ((K - 1) - q)
    h = IMG_H // 2
    o_ref[...] = a[:, :h] | (a[:, h:] << 16)


def _pack_addresses(pro, tgt):
    return pl.pallas_call(
        _pack_body,
        grid=(NB,),
        in_specs=[
            pl.BlockSpec((1, IMG_H, IMG_W), lambda i: (i, 0, 0)),
            pl.BlockSpec((1, IMG_H, IMG_W), lambda i: (i, 0, 0)),
        ],
        out_specs=pl.BlockSpec((1, IMG_H // 2, IMG_W), lambda i: (i, 0, 0)),
        out_shape=jax.ShapeDtypeStruct((NB, IMG_H // 2, IMG_W), jnp.int32),
    )(pro, tgt)


def _sc_body(addr_hbm, out_hbm,
             hist16, buf, myhist, comb, outbuf, shared,
             sem0, sem1):
    c = lax.axis_index("c")
    s = lax.axis_index("s")
    img = c * (NB // NC) + s // TILES_PER_IMG
    sub = s % TILES_PER_IMG
    base = pl.multiple_of(img * WORDS_PER_IMG + sub * WCHUNK, WPIECE)

    lane = lax.iota(jnp.int32, L)
    lane_hb = lane * HB
    ones = jnp.full((L,), 1.0, jnp.float32)
    zvec = jnp.zeros((L,), jnp.float32)

    @plsc.parallel_loop(0, (L * HB) // L, 1, unroll=8)
    def _zero(i):
        hist16[pl.ds(i * L, L)] = zvec

    sems = (sem0, sem1)

    def start(piece, b):
        off = pl.multiple_of(base + piece * WPIECE, WPIECE)
        return pltpu.async_copy(addr_hbm.at[pl.ds(off, WPIECE)],
                                buf.at[b], sems[b])

    handle = start(0, 0)
    for piece in range(NPIECE):
        cur = piece & 1
        handle.wait()
        if piece + 1 < NPIECE:
            handle = start(piece + 1, 1 - cur)

        @plsc.parallel_loop(0, VEC_PER_PIECE, 1, unroll=UNROLL)
        def _scat(v):
            w = buf[cur, pl.ds(v * L, L)]
            lo = w & 0xFFFF
            hi = jnp.right_shift(w, 16)
            plsc.addupdate_scatter(hist16, [lane_hb + lo], ones)
            plsc.addupdate_scatter(hist16, [lane_hb + hi], ones)

    @plsc.parallel_loop(0, HB // L, 1, unroll=2)
    def _lred(v):
        acc = hist16[pl.ds(v * L, L)]
        for ln in range(1, L):
            acc = acc + hist16[pl.ds(ln * HB + v * L, L)]
        myhist[pl.ds(v * L, L)] = acc

    pltpu.sync_copy(myhist, shared.at[s])
    plsc.subcore_barrier()

    @pl.when(sub == 0)
    def _owner():
        for r in range(TILES_PER_IMG):
            pltpu.sync_copy(shared.at[s + r], comb.at[r])

        def g_body(v, accv):
            gv = comb[0, pl.ds(K + v * L, L)]
            for r in range(1, TILES_PER_IMG):
                gv = gv + comb[r, pl.ds(K + v * L, L)]
            return accv + gv
        g_vec = lax.fori_loop(0, K // L, g_body, zvec)
        G = jnp.sum(g_vec)

        def scan_body(v, carry):
            cn, cb, accv = carry
            bgv = comb[0, pl.ds(v * L, L)]
            fgv = comb[0, pl.ds(K + v * L, L)]
            for r in range(1, TILES_PER_IMG):
                bgv = bgv + comb[r, pl.ds(v * L, L)]
                fgv = fgv + comb[r, pl.ds(K + v * L, L)]
            hn = bgv + fgv
            cumn = plsc.cumsum(hn) + cn
            cumb = plsc.cumsum(bgv) + cb
            accv = accv + cumn / (G + cumb)
            return (cn + jnp.sum(hn), cb + jnp.sum(bgv), accv)

        cn, cb, accv = lax.fori_loop(
            0, K // L, scan_body,
            (jnp.float32(0.0), jnp.float32(0.0), zvec))
        h = 1.0 / float(K)
        loss = h * jnp.sum(accv) - 0.5 * h
        outbuf[...] = jnp.where(lane == 0, loss, 0.0)
        pltpu.sync_copy(outbuf, out_hbm.at[img])


def _sc_losses(addr_flat):
    mesh = plsc.VectorSubcoreMesh(core_axis_name="c", subcore_axis_name="s",
                                  num_cores=NC, num_subcores=NS)
    return pl.kernel(
        _sc_body,
        out_type=jax.ShapeDtypeStruct((NB, L), jnp.float32),
        mesh=mesh,
        compiler_params=pltpu.CompilerParams(needs_layout_passes=False),
        scratch_types=[
            pltpu.VMEM((L * HB,), jnp.float32),
            pltpu.VMEM((2, WPIECE), jnp.int32),
            pltpu.VMEM((HB,), jnp.float32),
            pltpu.VMEM((TILES_PER_IMG, HB), jnp.float32),
            pltpu.VMEM((L,), jnp.float32),
            pltpu.VMEM_SHARED((NS, HB), jnp.float32),
            pltpu.SemaphoreType.DMA,
            pltpu.SemaphoreType.DMA,
        ],
    )(addr_flat)


def _mean_body(x_ref, o_ref):
    o_ref[...] = jnp.sum(x_ref[...], keepdims=True).reshape(1, 1) * (1.0 / NB)


def kernel(outputs, targets):
    packed = _pack_addresses(outputs, targets.astype(jnp.int32)).reshape(-1)
    losses = _sc_losses(packed)
    out = pl.pallas_call(
        _mean_body,
        out_shape=jax.ShapeDtypeStruct((1, 1), jnp.float32),
    )(losses)
    return out[0, 0]


# speedup vs baseline: 1.0331x; 1.0331x over previous
import functools

import jax
import jax.numpy as jnp
from jax import lax
from jax.experimental import pallas as pl
from jax.experimental.pallas import tpu as pltpu
from jax.experimental.pallas import tpu_sc as plsc

NC = 2
NS = 16
L = 16
K = 1024
NB = 8
NPIX = 512 * 512
TILES_PER_IMG = (NC * NS) // NB
WORDS_PER_IMG = NPIX // 2
WCHUNK = WORDS_PER_IMG // TILES_PER_IMG
WPIECE = 4096
NPIECE = WCHUNK // WPIECE
VEC_PER_PIECE = WPIECE // L
HB = 2 * K
UNROLL = 8

IMG_H = 512
IMG_W = 512


def _pack_body(p_ref, t_ref, o_ref):
    p = p_ref[...]
    t = t_ref[...]
    e = jnp.abs(t.astype(jnp.float32) - p)
    q = jnp.minimum((e * float(K)).astype(jnp.int32), K - 1)
    a = t * K + ((K - 1) - q)
    h = IMG_H // 2
    o_ref[...] = a[:, :h] | (a[:, h:] << 16)


def _pack_addresses(pro, tgt):
    return pl.pallas_call(
        _pack_body,
        grid=(NB,),
        in_specs=[
            pl.BlockSpec((1, IMG_H, IMG_W), lambda i: (i, 0, 0)),
            pl.BlockSpec((1, IMG_H, IMG_W), lambda i: (i, 0, 0)),
        ],
        out_specs=pl.BlockSpec((1, IMG_H // 2, IMG_W), lambda i: (i, 0, 0)),
        out_shape=jax.ShapeDtypeStruct((NB, IMG_H // 2, IMG_W), jnp.int32),
    )(pro, tgt)


def _sc_body(addr_hbm, out_hbm,
             hist16, hist16b, buf, myhist, comb, outbuf, shared,
             sem0, sem1):
    c = lax.axis_index("c")
    s = lax.axis_index("s")
    img = c * (NB // NC) + s // TILES_PER_IMG
    sub = s % TILES_PER_IMG
    base = pl.multiple_of(img * WORDS_PER_IMG + sub * WCHUNK, WPIECE)

    lane = lax.iota(jnp.int32, L)
    lane_hb = lane * HB
    ones = jnp.full((L,), 1.0, jnp.float32)
    zvec = jnp.zeros((L,), jnp.float32)

    @plsc.parallel_loop(0, (L * HB) // L, 1, unroll=8)
    def _zero(i):
        hist16[pl.ds(i * L, L)] = zvec
        hist16b[pl.ds(i * L, L)] = zvec

    sems = (sem0, sem1)

    def start(piece, b):
        off = pl.multiple_of(base + piece * WPIECE, WPIECE)
        return pltpu.async_copy(addr_hbm.at[pl.ds(off, WPIECE)],
                                buf.at[b], sems[b])

    handle = start(0, 0)
    for piece in range(NPIECE):
        cur = piece & 1
        handle.wait()
        if piece + 1 < NPIECE:
            handle = start(piece + 1, 1 - cur)

        @plsc.parallel_loop(0, VEC_PER_PIECE, 1, unroll=UNROLL)
        def _scat(v):
            w = buf[cur, pl.ds(v * L, L)]
            lo = w & 0xFFFF
            hi = jnp.right_shift(w, 16)
            plsc.addupdate_scatter(hist16, [lane_hb + lo], ones)
            plsc.addupdate_scatter(hist16b, [lane_hb + hi], ones)

    @plsc.parallel_loop(0, HB // L, 1, unroll=2)
    def _lred(v):
        acc = hist16[pl.ds(v * L, L)] + hist16b[pl.ds(v * L, L)]
        for ln in range(1, L):
            acc = acc + hist16[pl.ds(ln * HB + v * L, L)]
            acc = acc + hist16b[pl.ds(ln * HB + v * L, L)]
        myhist[pl.ds(v * L, L)] = acc

    pltpu.sync_copy(myhist, shared.at[s])
    plsc.subcore_barrier()

    @pl.when(sub == 0)
    def _owner():
        for r in range(TILES_PER_IMG):
            pltpu.sync_copy(shared.at[s + r], comb.at[r])

        def g_body(v, accv):
            gv = comb[0, pl.ds(K + v * L, L)]
            for r in range(1, TILES_PER_IMG):
                gv = gv + comb[r, pl.ds(K + v * L, L)]
            return accv + gv
        g_vec = lax.fori_loop(0, K // L, g_body, zvec)
        G = jnp.sum(g_vec)

        def scan_body(v, carry):
            cn, cb, accv = carry
            bgv = comb[0, pl.ds(v * L, L)]
            fgv = comb[0, pl.ds(K + v * L, L)]
            for r in range(1, TILES_PER_IMG):
                bgv = bgv + comb[r, pl.ds(v * L, L)]
                fgv = fgv + comb[r, pl.ds(K + v * L, L)]
            hn = bgv + fgv
            cumn = plsc.cumsum(hn) + cn
            cumb = plsc.cumsum(bgv) + cb
            accv = accv + cumn / (G + cumb)
            return (cn + jnp.sum(hn), cb + jnp.sum(bgv), accv)

        cn, cb, accv = lax.fori_loop(
            0, K // L, scan_body,
            (jnp.float32(0.0), jnp.float32(0.0), zvec))
        h = 1.0 / float(K)
        loss = h * jnp.sum(accv) - 0.5 * h
        outbuf[...] = jnp.where(lane == 0, loss, 0.0)
        pltpu.sync_copy(outbuf, out_hbm.at[img])


def _sc_losses(addr_flat):
    mesh = plsc.VectorSubcoreMesh(core_axis_name="c", subcore_axis_name="s",
                                  num_cores=NC, num_subcores=NS)
    return pl.kernel(
        _sc_body,
        out_type=jax.ShapeDtypeStruct((NB, L), jnp.float32),
        mesh=mesh,
        compiler_params=pltpu.CompilerParams(needs_layout_passes=False),
        scratch_types=[
            pltpu.VMEM((L * HB,), jnp.float32),
            pltpu.VMEM((L * HB,), jnp.float32),
            pltpu.VMEM((2, WPIECE), jnp.int32),
            pltpu.VMEM((HB,), jnp.float32),
            pltpu.VMEM((TILES_PER_IMG, HB), jnp.float32),
            pltpu.VMEM((L,), jnp.float32),
            pltpu.VMEM_SHARED((NS, HB), jnp.float32),
            pltpu.SemaphoreType.DMA,
            pltpu.SemaphoreType.DMA,
        ],
    )(addr_flat)


def _mean_body(x_ref, o_ref):
    o_ref[...] = jnp.sum(x_ref[...], keepdims=True).reshape(1, 1) * (1.0 / NB)


def kernel(outputs, targets):
    packed = _pack_addresses(outputs, targets.astype(jnp.int32)).reshape(-1)
    losses = _sc_losses(packed)
    out = pl.pallas_call(
        _mean_body,
        out_shape=jax.ShapeDtypeStruct((1, 1), jnp.float32),
    )(losses)
    return out[0, 0]


# speedup vs baseline: 1.0678x; 1.0336x over previous
import functools

import jax
import jax.numpy as jnp
from jax import lax
from jax.experimental import pallas as pl
from jax.experimental.pallas import tpu as pltpu
from jax.experimental.pallas import tpu_sc as plsc

NC = 2
NS = 16
L = 16
K = 1024
NB = 8
NPIX = 512 * 512
TILES_PER_IMG = (NC * NS) // NB
WORDS_PER_IMG = NPIX // 2
WCHUNK = WORDS_PER_IMG // TILES_PER_IMG
WPIECE = 4096
NPIECE = WCHUNK // WPIECE
VEC_PER_PIECE = WPIECE // L
HB = 2 * K
UNROLL = 8

IMG_H = 512
IMG_W = 512


def _pack_body(p_ref, t_ref, o_ref):
    p = p_ref[...]
    t = t_ref[...]
    e = jnp.abs(t.astype(jnp.float32) - p)
    q = jnp.minimum((e * float(K)).astype(jnp.int32), K - 1)
    a = t * K + ((K - 1) - q)
    h = IMG_H // 2
    o_ref[...] = a[:, :h] | (a[:, h:] << 16)


IMGS_PER_STEP = 2


def _pack_addresses(pro, tgt):
    return pl.pallas_call(
        _pack_body,
        grid=(NB // IMGS_PER_STEP,),
        in_specs=[
            pl.BlockSpec((IMGS_PER_STEP, IMG_H, IMG_W), lambda i: (i, 0, 0)),
            pl.BlockSpec((IMGS_PER_STEP, IMG_H, IMG_W), lambda i: (i, 0, 0)),
        ],
        out_specs=pl.BlockSpec((IMGS_PER_STEP, IMG_H // 2, IMG_W),
                               lambda i: (i, 0, 0)),
        out_shape=jax.ShapeDtypeStruct((NB, IMG_H // 2, IMG_W), jnp.int32),
    )(pro, tgt)


def _sc_body(addr_hbm, out_hbm,
             hist16, hist16b, buf, myhist, comb, outbuf, shared,
             sem0, sem1):
    c = lax.axis_index("c")
    s = lax.axis_index("s")
    img = c * (NB // NC) + s // TILES_PER_IMG
    sub = s % TILES_PER_IMG
    base = pl.multiple_of(img * WORDS_PER_IMG + sub * WCHUNK, WPIECE)

    lane = lax.iota(jnp.int32, L)
    lane_hb = lane * HB
    ones = jnp.full((L,), 1.0, jnp.float32)
    zvec = jnp.zeros((L,), jnp.float32)

    @plsc.parallel_loop(0, (L * HB) // L, 1, unroll=8)
    def _zero(i):
        hist16[pl.ds(i * L, L)] = zvec
        hist16b[pl.ds(i * L, L)] = zvec

    sems = (sem0, sem1)

    def start(piece, b):
        off = pl.multiple_of(base + piece * WPIECE, WPIECE)
        return pltpu.async_copy(addr_hbm.at[pl.ds(off, WPIECE)],
                                buf.at[b], sems[b])

    handle = start(0, 0)
    for piece in range(NPIECE):
        cur = piece & 1
        handle.wait()
        if piece + 1 < NPIECE:
            handle = start(piece + 1, 1 - cur)

        @plsc.parallel_loop(0, VEC_PER_PIECE, 1, unroll=UNROLL)
        def _scat(v):
            w = buf[cur, pl.ds(v * L, L)]
            lo = w & 0xFFFF
            hi = jnp.right_shift(w, 16)
            plsc.addupdate_scatter(hist16, [lane_hb + lo], ones)
            plsc.addupdate_scatter(hist16b, [lane_hb + hi], ones)

    @plsc.parallel_loop(0, HB // L, 1, unroll=2)
    def _lred(v):
        acc = hist16[pl.ds(v * L, L)] + hist16b[pl.ds(v * L, L)]
        for ln in range(1, L):
            acc = acc + hist16[pl.ds(ln * HB + v * L, L)]
            acc = acc + hist16b[pl.ds(ln * HB + v * L, L)]
        myhist[pl.ds(v * L, L)] = acc

    pltpu.sync_copy(myhist, shared.at[s])
    plsc.subcore_barrier()

    @pl.when(sub == 0)
    def _owner():
        for r in range(TILES_PER_IMG):
            pltpu.sync_copy(shared.at[s + r], comb.at[r])

        def g_body(v, accv):
            gv = comb[0, pl.ds(K + v * L, L)]
            for r in range(1, TILES_PER_IMG):
                gv = gv + comb[r, pl.ds(K + v * L, L)]
            return accv + gv
        g_vec = lax.fori_loop(0, K // L, g_body, zvec)
        G = jnp.sum(g_vec)

        def scan_body(v, carry):
            cn, cb, accv = carry
            bgv = comb[0, pl.ds(v * L, L)]
            fgv = comb[0, pl.ds(K + v * L, L)]
            for r in range(1, TILES_PER_IMG):
                bgv = bgv + comb[r, pl.ds(v * L, L)]
                fgv = fgv + comb[r, pl.ds(K + v * L, L)]
            hn = bgv + fgv
            cumn = plsc.cumsum(hn) + cn
            cumb = plsc.cumsum(bgv) + cb
            accv = accv + cumn / (G + cumb)
            return (cn + jnp.sum(hn), cb + jnp.sum(bgv), accv)

        cn, cb, accv = lax.fori_loop(
            0, K // L, scan_body,
            (jnp.float32(0.0), jnp.float32(0.0), zvec))
        h = 1.0 / float(K)
        loss = h * jnp.sum(accv) - 0.5 * h
        outbuf[...] = jnp.where(lane == 0, loss, 0.0)
        pltpu.sync_copy(outbuf, out_hbm.at[img])


def _sc_losses(addr_flat):
    mesh = plsc.VectorSubcoreMesh(core_axis_name="c", subcore_axis_name="s",
                                  num_cores=NC, num_subcores=NS)
    return pl.kernel(
        _sc_body,
        out_type=jax.ShapeDtypeStruct((NB, L), jnp.float32),
        mesh=mesh,
        compiler_params=pltpu.CompilerParams(needs_layout_passes=False),
        scratch_types=[
            pltpu.VMEM((L * HB,), jnp.float32),
            pltpu.VMEM((L * HB,), jnp.float32),
            pltpu.VMEM((2, WPIECE), jnp.int32),
            pltpu.VMEM((HB,), jnp.float32),
            pltpu.VMEM((TILES_PER_IMG, HB), jnp.float32),
            pltpu.VMEM((L,), jnp.float32),
            pltpu.VMEM_SHARED((NS, HB), jnp.float32),
            pltpu.SemaphoreType.DMA,
            pltpu.SemaphoreType.DMA,
        ],
    )(addr_flat)


def _mean_body(x_ref, o_ref):
    o_ref[...] = jnp.sum(x_ref[...], keepdims=True).reshape(1, 1) * (1.0 / NB)


def kernel(outputs, targets):
    packed = _pack_addresses(outputs, targets.astype(jnp.int32)).reshape(-1)
    losses = _sc_losses(packed)
    out = pl.pallas_call(
        _mean_body,
        out_shape=jax.ShapeDtypeStruct((1, 1), jnp.float32),
    )(losses)
    return out[0, 0]


# speedup vs baseline: 1.0792x; 1.0108x over previous
import functools

import jax
import jax.numpy as jnp
from jax import lax
from jax.experimental import pallas as pl
from jax.experimental.pallas import tpu as pltpu
from jax.experimental.pallas import tpu_sc as plsc

NC = 2
NS = 16
L = 16
K = 1024
NB = 8
NPIX = 512 * 512
TILES_PER_IMG = (NC * NS) // NB
WORDS_PER_IMG = NPIX // 2
WCHUNK = WORDS_PER_IMG // TILES_PER_IMG
WPIECE = 8192
NPIECE = WCHUNK // WPIECE
VEC_PER_PIECE = WPIECE // L
HB = 2 * K
UNROLL = 8

IMG_H = 512
IMG_W = 512


def _pack_body(p_ref, t_ref, o_ref):
    p = p_ref[...]
    t = t_ref[...]
    e = jnp.abs(t.astype(jnp.float32) - p)
    q = jnp.minimum((e * float(K)).astype(jnp.int32), K - 1)
    a = t * K + ((K - 1) - q)
    h = IMG_H // 2
    o_ref[...] = a[:, :h] | (a[:, h:] << 16)


IMGS_PER_STEP = 4


def _pack_addresses(pro, tgt):
    return pl.pallas_call(
        _pack_body,
        grid=(NB // IMGS_PER_STEP,),
        in_specs=[
            pl.BlockSpec((IMGS_PER_STEP, IMG_H, IMG_W), lambda i: (i, 0, 0)),
            pl.BlockSpec((IMGS_PER_STEP, IMG_H, IMG_W), lambda i: (i, 0, 0)),
        ],
        out_specs=pl.BlockSpec((IMGS_PER_STEP, IMG_H // 2, IMG_W),
                               lambda i: (i, 0, 0)),
        out_shape=jax.ShapeDtypeStruct((NB, IMG_H // 2, IMG_W), jnp.int32),
    )(pro, tgt)


def _sc_body(addr_hbm, out_hbm,
             hist16, hist16b, buf, myhist, comb, outbuf, shared,
             sem0, sem1):
    c = lax.axis_index("c")
    s = lax.axis_index("s")
    img = c * (NB // NC) + s // TILES_PER_IMG
    sub = s % TILES_PER_IMG
    base = pl.multiple_of(img * WORDS_PER_IMG + sub * WCHUNK, WPIECE)

    lane = lax.iota(jnp.int32, L)
    lane_hb = lane * HB
    ones = jnp.full((L,), 1.0, jnp.float32)
    zvec = jnp.zeros((L,), jnp.float32)

    @plsc.parallel_loop(0, (L * HB) // L, 1, unroll=8)
    def _zero(i):
        hist16[pl.ds(i * L, L)] = zvec
        hist16b[pl.ds(i * L, L)] = zvec

    sems = (sem0, sem1)

    def start(piece, b):
        off = pl.multiple_of(base + piece * WPIECE, WPIECE)
        return pltpu.async_copy(addr_hbm.at[pl.ds(off, WPIECE)],
                                buf.at[b], sems[b])

    handle = start(0, 0)
    for piece in range(NPIECE):
        cur = piece & 1
        handle.wait()
        if piece + 1 < NPIECE:
            handle = start(piece + 1, 1 - cur)

        @plsc.parallel_loop(0, VEC_PER_PIECE, 1, unroll=UNROLL)
        def _scat(v):
            w = buf[cur, pl.ds(v * L, L)]
            lo = w & 0xFFFF
            hi = jnp.right_shift(w, 16)
            plsc.addupdate_scatter(hist16, [lane_hb + lo], ones)
            plsc.addupdate_scatter(hist16b, [lane_hb + hi], ones)

    @plsc.parallel_loop(0, HB // L, 1, unroll=2)
    def _lred(v):
        acc = hist16[pl.ds(v * L, L)] + hist16b[pl.ds(v * L, L)]
        for ln in range(1, L):
            acc = acc + hist16[pl.ds(ln * HB + v * L, L)]
            acc = acc + hist16b[pl.ds(ln * HB + v * L, L)]
        myhist[pl.ds(v * L, L)] = acc

    pltpu.sync_copy(myhist, shared.at[s])
    plsc.subcore_barrier()

    @pl.when(sub == 0)
    def _owner():
        for r in range(TILES_PER_IMG):
            pltpu.sync_copy(shared.at[s + r], comb.at[r])

        def g_body(v, accv):
            gv = comb[0, pl.ds(K + v * L, L)]
            for r in range(1, TILES_PER_IMG):
                gv = gv + comb[r, pl.ds(K + v * L, L)]
            return accv + gv
        g_vec = lax.fori_loop(0, K // L, g_body, zvec)
        G = jnp.sum(g_vec)

        def scan_body(v, carry):
            cn, cb, accv = carry
            bgv = comb[0, pl.ds(v * L, L)]
            fgv = comb[0, pl.ds(K + v * L, L)]
            for r in range(1, TILES_PER_IMG):
                bgv = bgv + comb[r, pl.ds(v * L, L)]
                fgv = fgv + comb[r, pl.ds(K + v * L, L)]
            hn = bgv + fgv
            cumn = plsc.cumsum(hn) + cn
            cumb = plsc.cumsum(bgv) + cb
            accv = accv + cumn / (G + cumb)
            return (cn + jnp.sum(hn), cb + jnp.sum(bgv), accv)

        cn, cb, accv = lax.fori_loop(
            0, K // L, scan_body,
            (jnp.float32(0.0), jnp.float32(0.0), zvec))
        h = 1.0 / float(K)
        loss = h * jnp.sum(accv) - 0.5 * h
        outbuf[...] = jnp.where(lane == 0, loss, 0.0)
        pltpu.sync_copy(outbuf, out_hbm.at[img])


def _sc_losses(addr_flat):
    mesh = plsc.VectorSubcoreMesh(core_axis_name="c", subcore_axis_name="s",
                                  num_cores=NC, num_subcores=NS)
    return pl.kernel(
        _sc_body,
        out_type=jax.ShapeDtypeStruct((NB, L), jnp.float32),
        mesh=mesh,
        compiler_params=pltpu.CompilerParams(needs_layout_passes=False),
        scratch_types=[
            pltpu.VMEM((L * HB,), jnp.float32),
            pltpu.VMEM((L * HB,), jnp.float32),
            pltpu.VMEM((2, WPIECE), jnp.int32),
            pltpu.VMEM((HB,), jnp.float32),
            pltpu.VMEM((TILES_PER_IMG, HB), jnp.float32),
            pltpu.VMEM((L,), jnp.float32),
            pltpu.VMEM_SHARED((NS, HB), jnp.float32),
            pltpu.SemaphoreType.DMA,
            pltpu.SemaphoreType.DMA,
        ],
    )(addr_flat)


def _mean_body(x_ref, o_ref):
    o_ref[...] = jnp.sum(x_ref[...], keepdims=True).reshape(1, 1) * (1.0 / NB)


def kernel(outputs, targets):
    packed = _pack_addresses(outputs, targets.astype(jnp.int32)).reshape(-1)
    losses = _sc_losses(packed)
    out = pl.pallas_call(
        _mean_body,
        out_shape=jax.ShapeDtypeStruct((1, 1), jnp.float32),
    )(losses)
    return out[0, 0]
